# baseline (speedup 1.0000x reference)
import functools

import jax
import jax.numpy as jnp
from jax.experimental import pallas as pl

B = 1024
SIZE = 26
D = 4096
NUM_LEVELS = 100
D_TILE = 512


def _encode_kernel(x_ref, pos_ref, lev_ref, out_ref):
    idx = jnp.clip(jnp.floor(x_ref[...] * NUM_LEVELS), 0, NUM_LEVELS - 1)
    idx = idx.astype(jnp.int32)
    lev = lev_ref[...]
    iota = jax.lax.broadcasted_iota(jnp.int32, (1, NUM_LEVELS), 1)
    acc = jnp.zeros((B, D_TILE), jnp.int32)
    for s in range(SIZE):
        w = pos_ref[s, :][None, :]
        m_s = (lev + w - 2.0 * lev * w).astype(jnp.int8)
        onehot = (idx[:, s][:, None] == iota).astype(jnp.int8)
        acc = acc + jnp.dot(onehot, m_s, preferred_element_type=jnp.int32)
    out_ref[...] = acc.astype(jnp.float32)


@jax.jit
def kernel(x, position, levels):
    grid = (D // D_TILE,)
    return pl.pallas_call(
        _encode_kernel,
        grid=grid,
        in_specs=[
            pl.BlockSpec((B, SIZE), lambda j: (0, 0)),
            pl.BlockSpec((SIZE, D_TILE), lambda j: (0, j)),
            pl.BlockSpec((NUM_LEVELS, D_TILE), lambda j: (0, j)),
        ],
        out_specs=pl.BlockSpec((B, D_TILE), lambda j: (0, j)),
        out_shape=jax.ShapeDtypeStruct((B, D), jnp.float32),
    )(x, position, levels)


# speedup vs baseline: 1.8551x; 1.8551x over previous
import functools

import jax
import jax.numpy as jnp
from jax.experimental import pallas as pl

B = 1024
SIZE = 26
D = 4096
NUM_LEVELS = 100
D_TILE = 512


def _encode_kernel(x_ref, pos_ref, lev_ref, out_ref):
    idx = jnp.clip(jnp.floor(x_ref[...] * NUM_LEVELS), 0, NUM_LEVELS - 1)
    idx = idx.astype(jnp.int32)
    lev = lev_ref[...]
    iota = jax.lax.broadcasted_iota(jnp.int32, (1, 128), 1)
    zpad = jnp.zeros((128 - NUM_LEVELS, D_TILE), jnp.bfloat16)
    tabs, hots = [], []
    for s in range(SIZE):
        w = pos_ref[s, :][None, :]
        m_s = (lev + w - 2.0 * lev * w).astype(jnp.bfloat16)
        tabs.append(jnp.concatenate([m_s, zpad], axis=0))
        hots.append((idx[:, s][:, None] == iota).astype(jnp.bfloat16))
    table = jnp.concatenate(tabs, axis=0)
    onehot = jnp.concatenate(hots, axis=1)
    out_ref[...] = jnp.dot(onehot, table, preferred_element_type=jnp.float32)


@jax.jit
def kernel(x, position, levels):
    grid = (D // D_TILE,)
    return pl.pallas_call(
        _encode_kernel,
        grid=grid,
        in_specs=[
            pl.BlockSpec((B, SIZE), lambda j: (0, 0)),
            pl.BlockSpec((SIZE, D_TILE), lambda j: (0, j)),
            pl.BlockSpec((NUM_LEVELS, D_TILE), lambda j: (0, j)),
        ],
        out_specs=pl.BlockSpec((B, D_TILE), lambda j: (0, j)),
        out_shape=jax.ShapeDtypeStruct((B, D), jnp.float32),
    )(x, position, levels)
